# baseline (speedup 1.0000x reference)
import functools

import jax
import jax.numpy as jnp
from jax import lax
from jax.experimental import pallas as pl
from jax.experimental.pallas import tpu as pltpu
from jax.experimental.pallas import tpu_sc as plsc

B = 16384
D = 128
NUM_REL = 1000

_info = plsc.get_sparse_core_info()
NC, NS, L = _info.num_cores, _info.num_subcores, _info.num_lanes
NW = NC * NS
B_PER_W = B // NW
CHUNK = 128
N_CHUNKS = B_PER_W // CHUNK
GROUPS = CHUNK // L
DSL = D // L

_BITREV = [0, 8, 4, 12, 2, 10, 6, 14, 1, 9, 5, 13, 3, 11, 7, 15]


def _sc_kernel():
    mesh = plsc.VectorSubcoreMesh(core_axis_name="c", subcore_axis_name="s")

    @functools.partial(
        pl.kernel,
        mesh=mesh,
        out_type=jax.ShapeDtypeStruct((B,), jnp.float32),
        scratch_types=[
            pltpu.VMEM((B_PER_W,), jnp.int32),
            pltpu.VMEM((2, CHUNK, D), jnp.float32),
            pltpu.VMEM((2, CHUNK, D), jnp.float32),
            pltpu.VMEM((2, CHUNK, D), jnp.float32),
            pltpu.VMEM((2, CHUNK), jnp.float32),
            pltpu.SemaphoreType.DMA,
            pltpu.SemaphoreType.DMA,
            pltpu.SemaphoreType.DMA,
            pltpu.SemaphoreType.DMA,
            pltpu.SemaphoreType.DMA,
            pltpu.SemaphoreType.DMA,
        ],
    )
    def k(src_hbm, ids_hbm, dst_hbm, table_hbm, out_hbm,
          idx_v, rel_v, src_v, dst_v, out_v,
          gs0, ss0, ds0, gs1, ss1, ds1):
        wid = lax.axis_index("s") * NC + lax.axis_index("c")
        base = wid * B_PER_W
        lanes = lax.iota(jnp.int32, L)
        sems = [(gs0, ss0, ds0), (gs1, ss1, ds1)]
        bitrev = (((lanes & 1) << 3) | ((lanes & 2) << 1)
                  | ((lanes & 4) >> 1) | ((lanes & 8) >> 3))

        dnums = lax.GatherDimensionNumbers(
            offset_dims=(), collapsed_slice_dims=(0,), start_index_map=(0,))

        def lane_perm(x, perm):
            return lax.gather(
                x, perm[:, None], dimension_numbers=dnums, slice_sizes=(1,),
                mode=lax.GatherScatterMode.PROMISE_IN_BOUNDS)

        def fold(x, m):
            return x + lane_perm(x, jnp.bitwise_xor(lanes, m))

        def combine(a, b, m):
            return jnp.where((lanes & m) == 0, fold(a, m), fold(b, m))

        pltpu.sync_copy(ids_hbm.at[pl.ds(base, B_PER_W)], idx_v)

        def fire(c):
            bb = c % 2
            g, s, d = sems[bb]
            rb = base + c * CHUNK
            return (
                pltpu.async_copy(
                    table_hbm.at[idx_v.at[pl.ds(c * CHUNK, CHUNK)]],
                    rel_v.at[bb], g),
                pltpu.async_copy(src_hbm.at[pl.ds(rb, CHUNK)],
                                 src_v.at[bb], s),
                pltpu.async_copy(dst_hbm.at[pl.ds(rb, CHUNK)],
                                 dst_v.at[bb], d),
            )

        inflight = fire(0)

        for c in range(N_CHUNKS):
            bb = c % 2
            nxt = fire(c + 1) if c + 1 < N_CHUNKS else None
            for h in inflight:
                h.wait()
            inflight = nxt

            def row_acc(r):
                acc0 = acc1 = None
                for j in range(DSL):
                    sl = pl.ds(j * L, L)
                    p = (src_v[bb, r, sl]
                         * rel_v[bb, r, sl]
                         * dst_v[bb, r, sl])
                    if j % 2 == 0:
                        acc0 = p if acc0 is None else acc0 + p
                    else:
                        acc1 = p if acc1 is None else acc1 + p
                return acc0 + acc1

            def group_body(g, _):
                r0 = g * L

                def quad(q):
                    a = [row_acc(r0 + 4 * q + i) for i in range(4)]
                    return combine(combine(a[0], a[1], 8),
                                   combine(a[2], a[3], 8), 4)

                qs = [quad(q) for q in range(4)]
                res = combine(combine(qs[0], qs[1], 2),
                              combine(qs[2], qs[3], 2), 1)
                out_v[bb, pl.ds(r0, L)] = lane_perm(res, bitrev)
                return 0

            lax.fori_loop(0, GROUPS, group_body, 0)
            pltpu.sync_copy(out_v.at[bb],
                            out_hbm.at[pl.ds(base + c * CHUNK, CHUNK)])

    return k


_scorer = _sc_kernel()


@jax.jit
def kernel(src_emb, rel_ids, dst_emb, rel_emb_table):
    ids = rel_ids.astype(jnp.int32)
    return _scorer(src_emb, ids, dst_emb, rel_emb_table)


# speedup vs baseline: 1.8106x; 1.8106x over previous
import functools

import jax
import jax.numpy as jnp
from jax import lax
from jax.experimental import pallas as pl
from jax.experimental.pallas import tpu as pltpu
from jax.experimental.pallas import tpu_sc as plsc

B = 16384
D = 128
NUM_REL = 1000

_info = plsc.get_sparse_core_info()
NC, NS, L = _info.num_cores, _info.num_subcores, _info.num_lanes
NW = NC * NS
B_PER_W = B // NW
CHUNK = 128
N_CHUNKS = B_PER_W // CHUNK
GROUPS = CHUNK // L
DSL = D // L

_BITREV = [0, 8, 4, 12, 2, 10, 6, 14, 1, 9, 5, 13, 3, 11, 7, 15]


def _sc_kernel():
    mesh = plsc.VectorSubcoreMesh(core_axis_name="c", subcore_axis_name="s")

    @functools.partial(
        pl.kernel,
        mesh=mesh,
        out_type=jax.ShapeDtypeStruct((B,), jnp.float32),
        scratch_types=[
            pltpu.VMEM((B_PER_W,), jnp.int32),
            pltpu.VMEM((2, CHUNK, D), jnp.float32),
            pltpu.VMEM((2, CHUNK, D), jnp.float32),
            pltpu.VMEM((2, CHUNK, D), jnp.float32),
            pltpu.VMEM((2, CHUNK), jnp.float32),
            pltpu.SemaphoreType.DMA,
            pltpu.SemaphoreType.DMA,
            pltpu.SemaphoreType.DMA,
            pltpu.SemaphoreType.DMA,
            pltpu.SemaphoreType.DMA,
            pltpu.SemaphoreType.DMA,
        ],
    )
    def k(src_hbm, ids_hbm, dst_hbm, table_hbm, out_hbm,
          idx_v, rel_v, src_v, dst_v, out_v,
          gs0, ss0, ds0, gs1, ss1, ds1):
        wid = lax.axis_index("s") * NC + lax.axis_index("c")
        base = wid * B_PER_W
        lanes = lax.iota(jnp.int32, L)
        sems = [(gs0, ss0, ds0), (gs1, ss1, ds1)]
        bitrev = (((lanes & 1) << 3) | ((lanes & 2) << 1)
                  | ((lanes & 4) >> 1) | ((lanes & 8) >> 3))

        dnums = lax.GatherDimensionNumbers(
            offset_dims=(), collapsed_slice_dims=(0,), start_index_map=(0,))

        def lane_perm(x, perm):
            return lax.gather(
                x, perm[:, None], dimension_numbers=dnums, slice_sizes=(1,),
                mode=lax.GatherScatterMode.PROMISE_IN_BOUNDS)

        def fold(x, m):
            return x + lane_perm(x, jnp.bitwise_xor(lanes, m))

        def combine(a, b, m):
            return jnp.where((lanes & m) == 0, fold(a, m), fold(b, m))

        def lane_sum(x):
            for m in (8, 4, 2, 1):
                x = fold(x, m)
            return x

        pltpu.sync_copy(ids_hbm.at[pl.ds(base, B_PER_W)], idx_v)

        def fire(c):
            bb = c % 2
            g, s, d = sems[bb]
            rb = base + c * CHUNK
            return (
                pltpu.async_copy(
                    table_hbm.at[idx_v.at[pl.ds(c * CHUNK, CHUNK)]],
                    rel_v.at[bb], g),
                pltpu.async_copy(src_hbm.at[pl.ds(rb, CHUNK)],
                                 src_v.at[bb], s),
                pltpu.async_copy(dst_hbm.at[pl.ds(rb, CHUNK)],
                                 dst_v.at[bb], d),
            )

        inflight = fire(0)

        for c in range(N_CHUNKS):
            bb = c % 2
            nxt = fire(c + 1) if c + 1 < N_CHUNKS else None
            for h in inflight:
                h.wait()
            inflight = nxt

            def group_body(g, _):
                r0 = g * L

                def row_body(i, res):
                    r = r0 + i
                    acc0 = acc1 = None
                    for j in range(DSL):
                        sl = pl.ds(j * L, L)
                        p = (src_v[bb, r, sl]
                             * rel_v[bb, r, sl]
                             * dst_v[bb, r, sl])
                        if j % 2 == 0:
                            acc0 = p if acc0 is None else acc0 + p
                        else:
                            acc1 = p if acc1 is None else acc1 + p
                    tot = lane_sum(acc0 + acc1)
                    return jnp.where(lanes == i, tot, res)

                res = lax.fori_loop(0, L, row_body,
                                    jnp.zeros((L,), jnp.float32))
                out_v[bb, pl.ds(r0, L)] = res
                return 0

            lax.fori_loop(0, GROUPS, group_body, 0)
            pltpu.sync_copy(out_v.at[bb],
                            out_hbm.at[pl.ds(base + c * CHUNK, CHUNK)])

    return k


_scorer = _sc_kernel()


@jax.jit
def kernel(src_emb, rel_ids, dst_emb, rel_emb_table):
    ids = rel_ids.astype(jnp.int32)
    return _scorer(src_emb, ids, dst_emb, rel_emb_table)


# speedup vs baseline: 1.9743x; 1.0904x over previous
import functools

import jax
import jax.numpy as jnp
from jax import lax
from jax.experimental import pallas as pl
from jax.experimental.pallas import tpu as pltpu
from jax.experimental.pallas import tpu_sc as plsc

B = 16384
D = 128
NUM_REL = 1000

_info = plsc.get_sparse_core_info()
NC, NS, L = _info.num_cores, _info.num_subcores, _info.num_lanes
NW = NC * NS
B_PER_W = B // NW
CHUNK = 128
N_CHUNKS = B_PER_W // CHUNK
GROUPS = CHUNK // L
DSL = D // L

_BITREV = [0, 8, 4, 12, 2, 10, 6, 14, 1, 9, 5, 13, 3, 11, 7, 15]


def _sc_kernel():
    mesh = plsc.VectorSubcoreMesh(core_axis_name="c", subcore_axis_name="s")

    @functools.partial(
        pl.kernel,
        mesh=mesh,
        out_type=jax.ShapeDtypeStruct((B,), jnp.float32),
        scratch_types=[
            pltpu.VMEM((B_PER_W,), jnp.int32),
            pltpu.VMEM((2, CHUNK, D), jnp.float32),
            pltpu.VMEM((2, CHUNK, D), jnp.float32),
            pltpu.VMEM((2, CHUNK, D), jnp.float32),
            pltpu.VMEM((2, CHUNK), jnp.float32),
            pltpu.VMEM_SHARED((NUM_REL, D), jnp.float32),
            pltpu.SemaphoreType.DMA,
            pltpu.SemaphoreType.DMA,
            pltpu.SemaphoreType.DMA,
            pltpu.SemaphoreType.DMA,
            pltpu.SemaphoreType.DMA,
            pltpu.SemaphoreType.DMA,
        ],
    )
    def k(src_hbm, ids_hbm, dst_hbm, table_hbm, out_hbm,
          idx_v, rel_v, src_v, dst_v, out_v, table_sh,
          gs0, ss0, ds0, gs1, ss1, ds1):
        wid = lax.axis_index("s") * NC + lax.axis_index("c")
        base = wid * B_PER_W
        lanes = lax.iota(jnp.int32, L)
        sems = [(gs0, ss0, ds0), (gs1, ss1, ds1)]
        bitrev = (((lanes & 1) << 3) | ((lanes & 2) << 1)
                  | ((lanes & 4) >> 1) | ((lanes & 8) >> 3))

        dnums = lax.GatherDimensionNumbers(
            offset_dims=(), collapsed_slice_dims=(0,), start_index_map=(0,))

        def lane_perm(x, perm):
            return lax.gather(
                x, perm[:, None], dimension_numbers=dnums, slice_sizes=(1,),
                mode=lax.GatherScatterMode.PROMISE_IN_BOUNDS)

        def fold(x, m):
            return x + lane_perm(x, jnp.bitwise_xor(lanes, m))

        def combine(a, b, m):
            return jnp.where((lanes & m) == 0, fold(a, m), fold(b, m))

        def lane_sum(x):
            for m in (8, 4, 2, 1):
                x = fold(x, m)
            return x

        def fire_linear(c):
            bb = c % 2
            _, s, d = sems[bb]
            rb = base + c * CHUNK
            return (
                pltpu.async_copy(src_hbm.at[pl.ds(rb, CHUNK)],
                                 src_v.at[bb], s),
                pltpu.async_copy(dst_hbm.at[pl.ds(rb, CHUNK)],
                                 dst_v.at[bb], d),
            )

        def fire_gather(c):
            bb = c % 2
            return pltpu.async_copy(
                table_sh.at[idx_v.at[pl.ds(c * CHUNK, CHUNK)]],
                rel_v.at[bb], sems[bb][0])

        def fire(c):
            return (fire_gather(c),) + fire_linear(c)

        lin0 = fire_linear(0)
        pltpu.sync_copy(ids_hbm.at[pl.ds(base, B_PER_W)], idx_v)
        sid = lax.axis_index("s")

        @pl.when(sid < 15)
        def _():
            rslab = sid * 64
            pltpu.sync_copy(table_hbm.at[pl.ds(rslab, 64)],
                            table_sh.at[pl.ds(rslab, 64)])

        @pl.when(sid == 15)
        def _():
            pltpu.sync_copy(table_hbm.at[pl.ds(960, 40)],
                            table_sh.at[pl.ds(960, 40)])

        plsc.subcore_barrier()
        inflight = (fire_gather(0),) + lin0

        for c in range(N_CHUNKS):
            bb = c % 2
            nxt = fire(c + 1) if c + 1 < N_CHUNKS else None
            for h in inflight:
                h.wait()
            inflight = nxt

            def group_body(g, _):
                r0 = g * L

                def row_body(i, res):
                    r = r0 + i
                    acc0 = acc1 = None
                    for j in range(DSL):
                        sl = pl.ds(j * L, L)
                        p = (src_v[bb, r, sl]
                             * rel_v[bb, r, sl]
                             * dst_v[bb, r, sl])
                        if j % 2 == 0:
                            acc0 = p if acc0 is None else acc0 + p
                        else:
                            acc1 = p if acc1 is None else acc1 + p
                    tot = lane_sum(acc0 + acc1)
                    return jnp.where(lanes == i, tot, res)

                res = lax.fori_loop(0, L, row_body,
                                    jnp.zeros((L,), jnp.float32))
                out_v[bb, pl.ds(r0, L)] = res
                return 0

            lax.fori_loop(0, GROUPS, group_body, 0)
            pltpu.sync_copy(out_v.at[bb],
                            out_hbm.at[pl.ds(base + c * CHUNK, CHUNK)])

    return k


_scorer = _sc_kernel()


@jax.jit
def kernel(src_emb, rel_ids, dst_emb, rel_emb_table):
    ids = rel_ids.astype(jnp.int32)
    return _scorer(src_emb, ids, dst_emb, rel_emb_table)
